# baseline (speedup 1.0000x reference)
import functools

import jax
import jax.numpy as jnp
from jax import lax
from jax.experimental import pallas as pl
from jax.experimental.pallas import tpu as pltpu
from jax.experimental.pallas import tpu_sc as plsc

K = 512
Q = 20
C = 3
S = 151
E = S * S
ROW = C * E
EPAD = 22816
NTAIL = 1425
PFX = 16
KPT = 32


def _iota16():
    return lax.iota(jnp.int32, 16)


def _body(x2, db2, db3, out3,
          qpfx, kpfx, krow, qrow, grow, flags, flags_all, outidx,
          chunkbuf, myc, sem, flags_sh, chunks_sh):
    c = lax.axis_index("c")
    w = lax.axis_index("s")
    lanes = _iota16()

    kbase = pl.multiple_of(KPT * w, KPT)
    cps = []
    for q in range(Q):
        cps.append(pltpu.async_copy(
            x2.at[q, pl.ds(0, PFX)], qpfx.at[pl.ds(PFX * q, PFX)], sem))
    for i in range(KPT):
        cps.append(pltpu.async_copy(
            db2.at[kbase + i, pl.ds(0, PFX)],
            kpfx.at[pl.ds(PFX * i, PFX)], sem))
    for cp in cps:
        cp.wait()

    def full_verify(kg, q):
        pltpu.sync_copy(db2.at[kg, pl.ds(0, EPAD)], krow)
        pltpu.sync_copy(x2.at[q, pl.ds(0, EPAD)], qrow)

        def vbody(j, acc):
            kv = krow[pl.ds(16 * j, 16)]
            qv = qrow[pl.ds(16 * j, 16)]
            return acc | (kv != qv)

        acc = lax.fori_loop(0, NTAIL, vbody, jnp.zeros((16,), jnp.bool_))
        kv = krow[pl.ds(NTAIL * 16, 16)]
        qv = qrow[pl.ds(NTAIL * 16, 16)]
        acc = acc | ((kv != qv) & (lanes == 0))
        return jnp.logical_not(jnp.any(acc)).astype(jnp.int32)

    qvecs = [qpfx[pl.ds(PFX * q, PFX)] for q in range(Q)]

    def key_body(i, carry):
        fv0, fv1 = carry
        kv = kpfx[pl.ds(PFX * i, PFX)]
        kg = kbase + i
        found = jnp.int32(0)
        for q in range(Q):
            qv = qvecs[q]
            pref = jnp.all(kv == qv)
            do = pref & (found == 0)
            ver = lax.cond(do,
                           lambda q=q: full_verify(kg, q),
                           lambda: jnp.int32(0))
            found = found | ver
        lane = i & 15
        half = i >> 4
        upd = ((lanes == lane).astype(jnp.int32) * found)
        fv0 = fv0 | jnp.where(half == 0, upd, 0)
        fv1 = fv1 | jnp.where(half == 1, upd, 0)
        return fv0, fv1

    zero16 = jnp.zeros((16,), jnp.int32)
    fv0, fv1 = lax.fori_loop(0, KPT, key_body, (zero16, zero16))
    flags[pl.ds(0, 16)] = fv0
    flags[pl.ds(16, 16)] = fv1
    pltpu.sync_copy(flags, flags_sh.at[pl.ds(kbase, KPT)])
    plsc.subcore_barrier()

    @pl.when(w == 0)
    def _():
        pltpu.sync_copy(flags_sh, flags_all)
        neg1 = jnp.full((16,), -1, jnp.int32)
        outidx[pl.ds(0, 16)] = neg1
        outidx[pl.ds(16, 16)] = neg1

        def cbody(j, off):
            v = flags_all[pl.ds(16 * j, 16)]
            m = v != 0
            mi = m.astype(jnp.int32)
            pos = off + plsc.cumsum(mi) - 1
            pos = jnp.maximum(pos, 0)
            idxv = lanes + 16 * j
            plsc.store_scatter(outidx, [pos], idxv, mask=m)
            return off + jnp.sum(mi)

        lax.fori_loop(0, K // 16, cbody, jnp.int32(0))

        i0 = outidx[pl.ds(0, 16)]
        i1 = outidx[pl.ds(16, 16)]
        g0 = jnp.minimum(i0 + 1, K - 1)
        g1 = jnp.minimum(i1 + 1, K - 1)
        half_chunks = (3 * Q) // 2
        for ch in range(C):
            for (g, qoff) in ((g0, 0), (g1, 16)):
                p = 3 * (lanes + qoff) + ch
                l = p - half_chunks * c
                valid = (l >= 0) & (l < half_chunks) & (lanes + qoff < Q)
                lc = jnp.clip(l, 0, half_chunks - 1)
                plsc.store_scatter(chunkbuf, [lc >> 1, lc & 1],
                                   3 * g + ch, mask=valid)
        pltpu.sync_copy(chunkbuf, chunks_sh)

    plsc.subcore_barrier()

    @pl.when(w < (3 * Q) // (2 * 2))
    def _():
        pltpu.sync_copy(chunks_sh.at[w], myc)
        myv = myc[pl.ds(0, 16)]
        for j in range(2):
            r = myv[j]
            orow = ((3 * Q) // 2) * c + 2 * w + j
            pltpu.sync_copy(db3.at[r], grow)
            pltpu.sync_copy(grow, out3.at[orow])


@jax.jit
def kernel(x, stacked_windows_3d):
    x2 = x.reshape(Q, ROW)
    db2 = stacked_windows_3d.reshape(K, ROW)
    db3 = stacked_windows_3d.reshape(K * C, E)

    mesh = plsc.VectorSubcoreMesh(core_axis_name="c", subcore_axis_name="s")
    f32 = jnp.float32
    i32 = jnp.int32
    run = pl.kernel(
        _body,
        out_type=jax.ShapeDtypeStruct((Q * C, E), f32),
        mesh=mesh,
        compiler_params=pltpu.CompilerParams(
            use_tc_tiling_on_sc=False, needs_layout_passes=False),
        scratch_types=[
            pltpu.VMEM((Q * PFX,), f32),
            pltpu.VMEM((KPT * PFX,), f32),
            pltpu.VMEM((EPAD,), f32),
            pltpu.VMEM((EPAD,), f32),
            pltpu.VMEM((E,), f32),
            pltpu.VMEM((KPT,), i32),
            pltpu.VMEM((K,), i32),
            pltpu.VMEM((K,), i32),
            pltpu.VMEM((16, 16), i32),
            pltpu.VMEM((16,), i32),
            pltpu.SemaphoreType.DMA,
            pltpu.VMEM_SHARED((K,), i32),
            pltpu.VMEM_SHARED((16, 16), i32),
        ],
    )
    out3 = run(x2, db2, db3)
    return out3.reshape(Q, C, S, S)


# speedup vs baseline: 7.6967x; 7.6967x over previous
import jax
import jax.numpy as jnp
from jax import lax
from jax.experimental import pallas as pl
from jax.experimental.pallas import tpu as pltpu
from jax.experimental.pallas import tpu_sc as plsc

K = 512
Q = 20
C = 3
S = 151
PFX = 16
KPT = 32
COLS = tuple(range(0, S - PFX, PFX)) + (S - PFX,)


def _iota16():
    return lax.iota(jnp.int32, 16)


def _body(x4, db4, px, pk, out4,
          qpfx, kpfx, krow, qrow, grow, flags, flags_all, outidx,
          oidx, flags_sh, outidx_sh):
    c = lax.axis_index("c")
    w = lax.axis_index("s")
    lanes = _iota16()

    kbase = pl.multiple_of(KPT * w, KPT)
    pltpu.sync_copy(px, qpfx)
    pltpu.sync_copy(pk.at[pl.ds(kbase, KPT), :], kpfx)

    def full_verify(kg, q):
        pltpu.sync_copy(db4.at[kg, 0], krow)
        pltpu.sync_copy(x4.at[q, 0], qrow)

        def vbody(r, acc):
            for col in COLS:
                acc = acc | (krow[r, pl.ds(col, PFX)]
                             != qrow[r, pl.ds(col, PFX)])
            return acc

        acc = lax.fori_loop(0, S, vbody, jnp.zeros((16,), jnp.bool_))
        return jnp.logical_not(jnp.any(acc)).astype(jnp.int32)

    qvecs = [qpfx[q, pl.ds(0, PFX)] for q in range(Q)]

    def key_body(i, carry):
        fv0, fv1 = carry
        kv = kpfx[i, pl.ds(0, PFX)]
        kg = kbase + i
        found = jnp.int32(0)
        for q in range(Q):
            pref = jnp.all(kv == qvecs[q])
            do = pref & (found == 0)
            ver = lax.cond(do,
                           lambda q=q: full_verify(kg, q),
                           lambda: jnp.int32(0))
            found = found | ver
        lane = i & 15
        half = i >> 4
        upd = ((lanes == lane).astype(jnp.int32) * found)
        fv0 = fv0 | jnp.where(half == 0, upd, 0)
        fv1 = fv1 | jnp.where(half == 1, upd, 0)
        return fv0, fv1

    zero16 = jnp.zeros((16,), jnp.int32)
    fv0, fv1 = lax.fori_loop(0, KPT, key_body, (zero16, zero16))
    flags[pl.ds(0, 16)] = fv0
    flags[pl.ds(16, 16)] = fv1
    pltpu.sync_copy(flags, flags_sh.at[pl.ds(kbase, KPT)])
    plsc.subcore_barrier()

    @pl.when(w == 0)
    def _():
        pltpu.sync_copy(flags_sh, flags_all)
        neg1 = jnp.full((16,), -1, jnp.int32)
        outidx[pl.ds(0, 16)] = neg1
        outidx[pl.ds(16, 16)] = neg1

        def cbody(j, off):
            v = flags_all[pl.ds(16 * j, 16)]
            m = v != 0
            mi = m.astype(jnp.int32)
            pos = off + plsc.cumsum(mi) - 1
            pos = jnp.maximum(pos, 0)
            idxv = lanes + 16 * j
            plsc.store_scatter(outidx, [pos], idxv, mask=m)
            return off + jnp.sum(mi)

        lax.fori_loop(0, K // 16, cbody, jnp.int32(0))
        pltpu.sync_copy(outidx.at[pl.ds(0, 32)], outidx_sh)

    plsc.subcore_barrier()

    @pl.when(w < (3 * Q) // (2 * 2))
    def _():
        pltpu.sync_copy(outidx_sh, oidx)
        g0 = jnp.minimum(oidx[pl.ds(0, 16)] + 1, K - 1)
        g1 = jnp.minimum(oidx[pl.ds(16, 16)] + 1, K - 1)
        for j in range(2):
            p = ((3 * Q) // 2) * c + 2 * w + j
            oq = (p * 86) >> 8
            och = p - 3 * oq
            g = (jnp.sum(jnp.where(lanes == oq, g0, 0))
                 + jnp.sum(jnp.where(lanes == oq - 16, g1, 0)))
            pltpu.sync_copy(db4.at[g, och], grow)
            pltpu.sync_copy(grow, out4.at[oq, och])


@jax.jit
def kernel(x, stacked_windows_3d):
    px = x[:, 0, 0, :PFX]
    pk = stacked_windows_3d[:, 0, 0, :PFX]

    mesh = plsc.VectorSubcoreMesh(core_axis_name="c", subcore_axis_name="s")
    f32 = jnp.float32
    i32 = jnp.int32
    run = pl.kernel(
        _body,
        out_type=jax.ShapeDtypeStruct((Q, C, S, S), f32),
        mesh=mesh,
        compiler_params=pltpu.CompilerParams(needs_layout_passes=False),
        scratch_types=[
            pltpu.VMEM((Q, PFX), f32),
            pltpu.VMEM((KPT, PFX), f32),
            pltpu.VMEM((S, S), f32),
            pltpu.VMEM((S, S), f32),
            pltpu.VMEM((S, S), f32),
            pltpu.VMEM((KPT,), i32),
            pltpu.VMEM((K,), i32),
            pltpu.VMEM((K,), i32),
            pltpu.VMEM((32,), i32),
            pltpu.VMEM_SHARED((K,), i32),
            pltpu.VMEM_SHARED((32,), i32),
        ],
    )
    return run(x, stacked_windows_3d, px, pk)


# speedup vs baseline: 10.1554x; 1.3195x over previous
import jax
import jax.numpy as jnp
from jax import lax
from jax.experimental import pallas as pl
from jax.experimental.pallas import tpu as pltpu
from jax.experimental.pallas import tpu_sc as plsc

K = 512
Q = 20
C = 3
S = 151
PFX = 16
MAXP = 256
LCAP = 32


def _body(x_t, db_t, qsplat_in, out_t,
          kpfx, qsplat, kploc, qploc, cnt_loc, pbuf, qbuf,
          kpair_v, qpair_v, np_v, acc_v, flags_all, outidx, oidx, tmp_v,
          acc_all, kall, qall, kploc_sh, qploc_sh, cnt_sh, kpair_sh,
          qpair_sh, np_sh, acc_sh, oidx_sh):
    c = lax.axis_index("c")
    w = lax.axis_index("s")
    lanes = lax.iota(jnp.int32, 16)
    i32 = jnp.int32

    pltpu.sync_copy(db_t.at[0, 0, pl.ds(0, PFX), :], kpfx)
    pltpu.sync_copy(qsplat_in, qsplat)
    npl = i32(0)
    for grp in range(2):
        off = 32 * w + 16 * grp
        kcols = [kpfx[e, pl.ds(off, 16)] for e in range(PFX)]
        for q in range(Q):
            m = kcols[0] == qsplat[pl.ds(16 * q, 16)]
            for e in range(1, PFX):
                m = m & (kcols[e] == qsplat[pl.ds(16 * (Q * e + q), 16)])
            mi = m.astype(i32)
            pos = npl + plsc.cumsum(mi) - 1
            posr = jnp.where(m, jnp.clip(pos, 0, LCAP - 1), LCAP + lanes)
            plsc.store_scatter(kploc, [posr], off + lanes, mask=m)
            plsc.store_scatter(qploc, [posr], jnp.full((16,), q, i32),
                               mask=m)
            npl = npl + jnp.sum(mi)
    cnt_loc[pl.ds(0, 16)] = jnp.full((16,), 1, i32) * jnp.minimum(npl, LCAP)
    pltpu.sync_copy(kploc, kploc_sh.at[w])
    pltpu.sync_copy(qploc, qploc_sh.at[w])
    pltpu.sync_copy(cnt_loc, cnt_sh.at[w])
    plsc.subcore_barrier()

    @pl.when(w == 0)
    def _():
        for t in range(16):
            pltpu.sync_copy(kploc_sh.at[t], kall.at[t])
            pltpu.sync_copy(qploc_sh.at[t], qall.at[t])
            pltpu.sync_copy(cnt_sh.at[t], tmp_v.at[t])
        off = i32(0)
        for t in range(16):
            cnt_t = tmp_v[t, pl.ds(0, 16)][0]
            for h in range(2):
                kv = kall[t, pl.ds(16 * h, 16)]
                qv = qall[t, pl.ds(16 * h, 16)]
                mh = (lanes + 16 * h) < cnt_t
                pos = off + 16 * h + lanes
                posr = jnp.where(mh, jnp.clip(pos, 0, MAXP - 1),
                                 MAXP + lanes)
                plsc.store_scatter(kpair_v, [posr], kv, mask=mh)
                plsc.store_scatter(qpair_v, [posr], qv, mask=mh)
            off = off + cnt_t
        np_v[pl.ds(0, 16)] = jnp.full((16,), 1, i32) * jnp.minimum(off, MAXP)
        pltpu.sync_copy(kpair_v, kpair_sh)
        pltpu.sync_copy(qpair_v, qpair_sh)
        pltpu.sync_copy(np_v, np_sh)
    plsc.subcore_barrier()

    pltpu.sync_copy(kpair_sh, kpair_v)
    pltpu.sync_copy(qpair_sh, qpair_v)
    pltpu.sync_copy(np_sh, np_v)
    npairs = np_v[pl.ds(0, 16)][0]
    npv = jnp.minimum((npairs + 15) >> 4, MAXP // 16)
    zero16 = jnp.zeros((16,), i32)
    for j in range(MAXP // 16):
        acc_v[pl.ds(16 * j, 16)] = zero16

    n_pl = (S - 1 - w) // 16 + 1

    def plane_body(i, _):
        s1 = w + 16 * i
        pltpu.sync_copy(db_t.at[0, s1], pbuf)
        pltpu.sync_copy(x_t.at[0, s1], qbuf)

        def pv_body(pv, _):
            kidx = jnp.clip(kpair_v[pl.ds(16 * pv, 16)], 0, K - 1)
            qidx = jnp.clip(qpair_v[pl.ds(16 * pv, 16)], 0, Q - 1)
            accv = acc_v[pl.ds(16 * pv, 16)]

            def s2_body(s2, accv):
                s2v = jnp.full((16,), s2, i32)
                kvec = plsc.load_gather(pbuf, [s2v, kidx])
                qvec = plsc.load_gather(qbuf, [s2v, qidx])
                return accv | (kvec != qvec).astype(i32)

            accv = lax.fori_loop(0, S, s2_body, accv)
            acc_v[pl.ds(16 * pv, 16)] = accv
            return 0

        lax.fori_loop(0, npv, pv_body, 0)
        return 0

    lax.fori_loop(0, n_pl, plane_body, 0)
    pltpu.sync_copy(acc_v, acc_sh.at[w])
    plsc.subcore_barrier()

    @pl.when(w == 0)
    def _():
        for t in range(16):
            pltpu.sync_copy(acc_sh.at[t], acc_all.at[t])
        for j in range((K + 32) // 16):
            flags_all[pl.ds(16 * j, 16)] = zero16
        for pv in range(MAXP // 16):
            orv = acc_all[0, pl.ds(16 * pv, 16)]
            for t in range(1, 16):
                orv = orv | acc_all[t, pl.ds(16 * pv, 16)]
            valid = (16 * pv + lanes) < npairs
            ok = valid & (orv == 0)
            kk = jnp.clip(kpair_v[pl.ds(16 * pv, 16)], 0, K - 1)
            posr = jnp.where(ok, kk, K + lanes)
            plsc.store_scatter(flags_all, [posr],
                               jnp.full((16,), 1, i32), mask=ok)
        neg1 = jnp.full((16,), -1, i32)
        outidx[pl.ds(0, 16)] = neg1
        outidx[pl.ds(16, 16)] = neg1

        def cbody(j, off):
            v = flags_all[pl.ds(16 * j, 16)]
            m = v != 0
            mi = m.astype(i32)
            pos = off + plsc.cumsum(mi) - 1
            pos = jnp.maximum(pos, 0)
            idxv = lanes + 16 * j
            plsc.store_scatter(outidx, [pos], idxv, mask=m)
            return off + jnp.sum(mi)

        lax.fori_loop(0, K // 16, cbody, i32(0))
        pltpu.sync_copy(outidx.at[pl.ds(0, 32)], oidx_sh)
    plsc.subcore_barrier()

    pltpu.sync_copy(oidx_sh, oidx)
    glo = jnp.minimum(oidx[pl.ds(0, 16)] + 1, K - 1)
    ghi = jnp.minimum(oidx[pl.ds(4, 16)] + 1, K - 1)
    W = 2 * w + c
    n_g = (S - 1 - W) // 32 + 1

    for ch in range(C):
        def gbody(i, _, ch=ch):
            s1 = W + 32 * i
            pltpu.sync_copy(db_t.at[ch, s1], pbuf)

            def s2_body(s2, _):
                s2v = jnp.full((16,), s2, i32)
                hi = plsc.load_gather(pbuf, [s2v, ghi])
                qbuf[s2, pl.ds(4, 16)] = hi
                lo = plsc.load_gather(pbuf, [s2v, glo])
                qbuf[s2, pl.ds(0, 16)] = lo
                return 0

            lax.fori_loop(0, S, s2_body, 0)
            pltpu.sync_copy(qbuf, out_t.at[ch, s1])
            return 0

        lax.fori_loop(0, n_g, gbody, 0)


@jax.jit
def kernel(x, stacked_windows_3d):
    f32 = jnp.float32
    i32 = jnp.int32
    x_t = jnp.transpose(x, (1, 2, 3, 0))
    db_t = jnp.transpose(stacked_windows_3d, (1, 2, 3, 0))
    qsplat_in = jnp.broadcast_to(
        x[:, 0, 0, :PFX].T[:, :, None], (PFX, Q, 16)).reshape(PFX * Q * 16)

    mesh = plsc.VectorSubcoreMesh(core_axis_name="c", subcore_axis_name="s")
    run = pl.kernel(
        _body,
        out_type=jax.ShapeDtypeStruct((C, S, S, Q), f32),
        mesh=mesh,
        compiler_params=pltpu.CompilerParams(needs_layout_passes=False),
        scratch_types=[
            pltpu.VMEM((PFX, K), f32),
            pltpu.VMEM((PFX * Q * 16,), f32),
            pltpu.VMEM((128,), i32),
            pltpu.VMEM((128,), i32),
            pltpu.VMEM((128,), i32),
            pltpu.VMEM((S, K), f32),
            pltpu.VMEM((S, Q), f32),
            pltpu.VMEM((MAXP + 16,), i32),
            pltpu.VMEM((MAXP + 16,), i32),
            pltpu.VMEM((16,), i32),
            pltpu.VMEM((MAXP,), i32),
            pltpu.VMEM((K + 32,), i32),
            pltpu.VMEM((K,), i32),
            pltpu.VMEM((32,), i32),
            pltpu.VMEM((16, 128), i32),
            pltpu.VMEM((16, MAXP), i32),
            pltpu.VMEM((16, 128), i32),
            pltpu.VMEM((16, 128), i32),
            pltpu.VMEM_SHARED((16, 128), i32),
            pltpu.VMEM_SHARED((16, 128), i32),
            pltpu.VMEM_SHARED((16, 128), i32),
            pltpu.VMEM_SHARED((MAXP + 16,), i32),
            pltpu.VMEM_SHARED((MAXP + 16,), i32),
            pltpu.VMEM_SHARED((16,), i32),
            pltpu.VMEM_SHARED((16, MAXP), i32),
            pltpu.VMEM_SHARED((32,), i32),
        ],
    )
    out_t = run(x_t, db_t, qsplat_in)
    return jnp.transpose(out_t, (3, 0, 1, 2))
